# Initial kernel scaffold; baseline (speedup 1.0000x reference)
import jax
import jax.numpy as jnp
from jax.experimental import pallas as pl


def kernel(input_ids, table):
    raise NotImplementedError("write your pallas kernel here")


# speedup vs baseline: 1.6828x; 1.6828x over previous
import functools

import jax
import jax.numpy as jnp
from jax import lax
from jax.experimental import pallas as pl
from jax.experimental.pallas import tpu as pltpu
from jax.experimental.pallas import tpu_sc as plsc

_NC = 2
_NS = 16
_NW = _NC * _NS

_VOCAB = 1000000
_OUT_DIM = 64
_CHUNK = 128


def _make_gather(n_rows: int):
    assert n_rows % (_NW * _CHUNK) == 0
    chunks_per_w = n_rows // (_NW * _CHUNK)

    mesh = plsc.VectorSubcoreMesh(core_axis_name="c", subcore_axis_name="s")

    @functools.partial(
        pl.kernel,
        out_type=jax.ShapeDtypeStruct((n_rows, _OUT_DIM), jnp.float32),
        mesh=mesh,
        scratch_types=[
            pltpu.VMEM((chunks_per_w, _CHUNK), jnp.int32),
            pltpu.VMEM((_CHUNK, _OUT_DIM), jnp.float32),
            pltpu.SemaphoreType.DMA,
        ],
        compiler_params=pltpu.CompilerParams(use_tc_tiling_on_sc=False),
    )
    def gather(table_hbm, idx_hbm, out_hbm, idx_v, rows_v, gsem):
        wid = lax.axis_index("s") * _NC + lax.axis_index("c")
        cbase = wid * chunks_per_w
        pltpu.sync_copy(idx_hbm.at[pl.ds(cbase, chunks_per_w)], idx_v)

        def step(j, carry):
            pltpu.async_copy(table_hbm.at[idx_v.at[j]], rows_v, gsem).wait()
            pltpu.sync_copy(
                rows_v, out_hbm.at[pl.ds((cbase + j) * _CHUNK, _CHUNK)]
            )
            return carry

        lax.fori_loop(0, chunks_per_w, step, 0)

    return gather


def kernel(input_ids, table):
    batch, hist = input_ids.shape
    n = batch * hist
    idx2d = input_ids.reshape(n // _CHUNK, _CHUNK).astype(jnp.int32)
    out = _make_gather(n)(table, idx2d)
    return out.reshape(batch, hist, _OUT_DIM)


# speedup vs baseline: 1.8851x; 1.1202x over previous
import functools

import jax
import jax.numpy as jnp
from jax import lax
from jax.experimental import pallas as pl
from jax.experimental.pallas import tpu as pltpu
from jax.experimental.pallas import tpu_sc as plsc

_NC = 2
_NS = 16
_NW = _NC * _NS

_OUT_DIM = 64
_CHUNK = 128
_NBUF = 8


def _make_gather(n_rows: int):
    assert n_rows % (_NW * _CHUNK * _NBUF) == 0
    chunks_per_w = n_rows // (_NW * _CHUNK)
    n_groups = chunks_per_w // _NBUF

    mesh = plsc.VectorSubcoreMesh(core_axis_name="c", subcore_axis_name="s")
    scratch = [pltpu.VMEM((chunks_per_w, _CHUNK), jnp.int32)]
    scratch += [pltpu.VMEM((_CHUNK, _OUT_DIM), jnp.float32)] * _NBUF
    scratch += [pltpu.SemaphoreType.DMA] * (2 * _NBUF)

    @functools.partial(
        pl.kernel,
        out_type=jax.ShapeDtypeStruct((n_rows, _OUT_DIM), jnp.float32),
        mesh=mesh,
        scratch_types=scratch,
        compiler_params=pltpu.CompilerParams(use_tc_tiling_on_sc=False),
    )
    def gather(table_hbm, idx_hbm, out_hbm, idx_v, *bufs):
        rows = bufs[:_NBUF]
        gsem = bufs[_NBUF : 2 * _NBUF]
        ssem = bufs[2 * _NBUF :]
        wid = lax.axis_index("s") * _NC + lax.axis_index("c")
        cbase = wid * chunks_per_w
        pltpu.sync_copy(idx_hbm.at[pl.ds(cbase, chunks_per_w)], idx_v)

        def gather_copy(j, b):
            return pltpu.make_async_copy(
                table_hbm.at[idx_v.at[j]], rows[b], gsem[b]
            )

        def store_copy(j, b):
            return pltpu.make_async_copy(
                rows[b], out_hbm.at[pl.ds((cbase + j) * _CHUNK, _CHUNK)], ssem[b]
            )

        for b in range(_NBUF):
            gather_copy(b, b).start()

        def group(g, carry):
            j0 = g * _NBUF
            for b in range(_NBUF):
                gather_copy(j0 + b, b).wait()
                store_copy(j0 + b, b).start()
            for b in range(_NBUF):
                store_copy(j0 + b, b).wait()
                gather_copy(j0 + _NBUF + b, b).start()
            return carry

        lax.fori_loop(0, n_groups - 1, group, 0)

        j0 = (n_groups - 1) * _NBUF
        for b in range(_NBUF):
            gather_copy(j0 + b, b).wait()
            store_copy(j0 + b, b).start()
        for b in range(_NBUF):
            store_copy(j0 + b, b).wait()

    return gather


def kernel(input_ids, table):
    batch, hist = input_ids.shape
    n = batch * hist
    idx2d = input_ids.reshape(n // _CHUNK, _CHUNK).astype(jnp.int32)
    out = _make_gather(n)(table, idx2d)
    return out.reshape(batch, hist, _OUT_DIM)
